# baseline (speedup 1.0000x reference)
import math

import jax
import jax.numpy as jnp
import numpy as np
from jax import lax
from jax.experimental import pallas as pl
from jax.experimental.pallas import tpu as pltpu
from jax.experimental.pallas import tpu_sc as plsc

_N = 50000
_HID = 48
_NC = 2
_NS = 16
_NW = _NC * _NS
_CH = 1568
_NP = _NW * _CH
_SUB = 112
_NSUB = _CH // _SUB
_NB = 65
_CNTW = 80

_R2_EDGES = tuple((k / 4.0) ** 2 for k in range(1, 8))


def _sc_pool_body(dx_h, dy_h, idx_h, hid_h, zacc_h, zcnt_h, oacc_h, ocnt_h,
                  dxv, dyv, idxv, binv, rowsv, cntv, shacc, sem1, sem2):
    cid = lax.axis_index("c")
    sid = lax.axis_index("s")
    wid = sid * _NC + cid

    pltpu.sync_copy(idx_h.at[wid], idxv)
    gathers = [
        pltpu.async_copy(hid_h.at[idxv.at[j]], rowsv.at[j], sem1)
        for j in range(_NSUB)
    ]

    @pl.when(sid == 0)
    def _():
        pltpu.sync_copy(zacc_h, shacc)

    pltpu.sync_copy(dx_h.at[wid], dxv)
    pltpu.sync_copy(dy_h.at[wid], dyv)
    pltpu.sync_copy(zcnt_h, cntv)

    ones = jnp.ones((16,), jnp.float32)

    def row_body(r, carry):
        for q in range(_SUB // 16):
            off = r * _SUB + q * 16
            vdx = dxv[pl.ds(off, 16)]
            vdy = dyv[pl.ds(off, 16)]
            d2 = vdx * vdx + vdy * vdy
            lt1 = (vdx < vdy).astype(jnp.int32)
            le0 = (vdx <= 0.0).astype(jnp.int32)
            lt0 = (vdx < 0.0).astype(jnp.int32)
            le3 = (vdx <= -vdy).astype(jnp.int32)
            lt3 = (vdx < -vdy).astype(jnp.int32)
            sat = ((vdx < 0.0)
                   & (d2 <= vdx * vdx * 1.0000001192092896)).astype(jnp.int32)
            vb = jnp.where(vdy < 0.0, 7 - (lt3 + lt0 + lt1),
                           lt1 + le0 + le3 + sat)
            ub = (d2 >= _R2_EDGES[0]).astype(jnp.int32)
            for t in _R2_EDGES[1:]:
                ub = ub + (d2 >= t).astype(jnp.int32)
            b16 = jnp.where(d2 <= 4.0, ub * 8 + vb, _NB - 1)
            binv[r, pl.ds(q * 16, 16)] = b16
            plsc.addupdate_scatter(cntv, [b16], ones)
        return carry

    lax.fori_loop(0, _NSUB, row_body, None)

    plsc.subcore_barrier()
    scatters = []
    for j in range(_NSUB):
        gathers[j].wait()
        scatters.append(
            pltpu.async_copy(rowsv.at[j], shacc.at[binv.at[j]], sem2,
                             add=True))
    for s in scatters:
        s.wait()
    pltpu.sync_copy(cntv, ocnt_h.at[wid])
    plsc.subcore_barrier()

    @pl.when(sid == 0)
    def _():
        pltpu.sync_copy(shacc, oacc_h.at[cid])


_sc_pool = pl.kernel(
    _sc_pool_body,
    out_type=(
        jax.ShapeDtypeStruct((_NC, _NB, _HID), jnp.float32),
        jax.ShapeDtypeStruct((_NW, _CNTW), jnp.float32),
    ),
    mesh=plsc.VectorSubcoreMesh(core_axis_name="c", subcore_axis_name="s",
                                num_cores=_NC, num_subcores=_NS),
    compiler_params=pltpu.CompilerParams(needs_layout_passes=False,
                                         use_tc_tiling_on_sc=False),
    scratch_types=[
        pltpu.VMEM((_CH,), jnp.float32),
        pltpu.VMEM((_CH,), jnp.float32),
        pltpu.VMEM((_NSUB, _SUB), jnp.int32),
        pltpu.VMEM((_NSUB, _SUB), jnp.int32),
        pltpu.VMEM((_NSUB, _SUB, _HID), jnp.float32),
        pltpu.VMEM((_CNTW,), jnp.float32),
        pltpu.VMEM_SHARED((_NB, _HID), jnp.float32),
        pltpu.SemaphoreType.DMA,
        pltpu.SemaphoreType.DMA,
    ],
)

_REP = np.equal(np.arange(3072)[None, :] // _HID,
                np.arange(64)[:, None]).astype(np.float32)


def _tail_body(p2_ref, pc_ref, rep_ref, wfc_ref, bfc_ref, img_ref, la_ref,
               fv_ref, h_ref, wza_ref, wzv_ref, wzf_ref, whz_ref, zb_ref,
               wna_ref, wnv_ref, wnf_ref, whn_ref, nb_ref, out_ref):
    accf = jnp.sum(p2_ref[...], axis=0, keepdims=True)
    cnt = jnp.sum(pc_ref[...], axis=0, keepdims=True)[:, :64]
    cnt_e = lax.dot_general(cnt, rep_ref[...],
                            (((1,), (0,)), ((), ())),
                            preferred_element_type=jnp.float32)
    scale = jnp.where(cnt_e > 1.0, 1.0 / jnp.maximum(cnt_e, 1.0), 1.0)
    spf = accf * scale
    fsp = lax.dot_general(spf, wfc_ref[...],
                          (((1,), (1,)), ((), ())),
                          preferred_element_type=jnp.float32)
    fsp = jnp.maximum(fsp + bfc_ref[...], 0.0)

    la = la_ref[...]
    v_img = la[0, 0].astype(jnp.int32)
    u_img = 40 - la[0, 1].astype(jnp.int32)
    mu = lax.broadcasted_iota(jnp.int32, (80, 80), 0) == u_img
    mv = lax.broadcasted_iota(jnp.int32, (80, 80), 1) == v_img
    m2 = (mu & mv).astype(jnp.float32)
    t = jnp.sum(img_ref[0] * m2[None, :, :], axis=2)
    fa = jnp.sum(t, axis=1, keepdims=True)

    fv = fv_ref[...]
    h = h_ref[...]
    zx = (jnp.sum(fa * wza_ref[...]) + jnp.sum(fv * wzv_ref[...])
          + jnp.sum(fsp * wzf_ref[...]) + jnp.sum(h * whz_ref[...])
          + zb_ref[0, 0])
    nx = (jnp.sum(fa * wna_ref[...]) + jnp.sum(fv * wnv_ref[...])
          + jnp.sum(fsp * wnf_ref[...]) + jnp.sum(h * whn_ref[...])
          + nb_ref[0, 0])
    zt = 1.0 / (1.0 + jnp.exp(-zx))
    nt = jnp.tanh(nx)
    out_ref[...] = (1.0 - zt) * nt + zt * h


def kernel(loc_agent, loc_others, loc_other_index, feature_img, f_vel,
           hiddens, hidden, W_fc, b_fc, weight_ir, weight_hr, bias_ir,
           bias_hr, weight_iz, weight_hz, bias_iz, bias_hz, weight_in,
           weight_hn, bias_in, bias_hn):
    f32 = jnp.float32
    dx = loc_others[:, 0] - loc_agent[0]
    dy = loc_others[:, 1] - loc_agent[1]
    pad = _NP - _N
    big = jnp.full((pad,), 3.0e9, f32)
    dx2 = jnp.concatenate([dx, big]).reshape(_NW, _CH)
    dy2 = jnp.concatenate([dy, big]).reshape(_NW, _CH)
    idx3 = jnp.concatenate([
        loc_other_index.astype(jnp.int32), jnp.zeros((pad,), jnp.int32)
    ]).reshape(_NW, _NSUB, _SUB)
    zacc = jnp.zeros((_NB, _HID), f32)
    zcnt = jnp.zeros((_CNTW,), f32)

    rt_zero = loc_agent[0] * 0.0
    hid1 = lax.optimization_barrier(hiddens.reshape(_N * _HID) + rt_zero)
    hid2 = hid1.reshape(_N, _HID)

    part_acc, part_cnt = _sc_pool(dx2, dy2, idx3, hid2, zacc, zcnt)
    p2 = part_acc.reshape(_NC, _NB * _HID)[:, :64 * _HID]

    img2 = feature_img
    la2 = loc_agent.reshape(1, 2)
    fv2 = f_vel.reshape(1, 16)
    h2 = hidden.reshape(1, _HID)
    wza = weight_iz[:32].reshape(32, 1)
    wzv = weight_iz[32:48].reshape(1, 16)
    wzf = weight_iz[48:].reshape(1, _HID)
    wna = weight_in[:32].reshape(32, 1)
    wnv = weight_in[32:48].reshape(1, 16)
    wnf = weight_in[48:].reshape(1, _HID)
    whz = weight_hz.reshape(1, _HID)
    whn = weight_hn.reshape(1, _HID)
    zb = (bias_iz + bias_hz).reshape(1, 1)
    nb = (bias_in + bias_hn).reshape(1, 1)
    bfc2 = b_fc.reshape(1, _HID)

    ht = pl.pallas_call(
        _tail_body,
        out_shape=jax.ShapeDtypeStruct((1, _HID), jnp.float32),
    )(p2, part_cnt, jnp.asarray(_REP), W_fc, bfc2, img2, la2, fv2, h2,
      wza, wzv, wzf, whz, zb, wna, wnv, wnf, whn, nb)
    return ht.reshape(_HID)


# speedup vs baseline: 1.0119x; 1.0119x over previous
import math

import jax
import jax.numpy as jnp
import numpy as np
from jax import lax
from jax.experimental import pallas as pl
from jax.experimental.pallas import tpu as pltpu
from jax.experimental.pallas import tpu_sc as plsc

_N = 50000
_HID = 48
_NC = 2
_NS = 16
_NW = _NC * _NS
_CH = 1568
_NP = _NW * _CH
_SUB = 112
_NSUB = _CH // _SUB
_NB = 65
_CNTW = 80

_R2_EDGES = tuple((k / 4.0) ** 2 for k in range(1, 8))


def _sc_pool_body(dx_h, dy_h, idx_h, hid_h, zacc_h, zcnt_h, oacc_h, ocnt_h,
                  dxv, dyv, idxv, binv, rowsv, cntv, shacc, sem1, sem2):
    cid = lax.axis_index("c")
    sid = lax.axis_index("s")
    wid = sid * _NC + cid

    pltpu.sync_copy(idx_h.at[wid], idxv)
    gathers = [
        pltpu.async_copy(hid_h.at[idxv.at[j]], rowsv.at[j], sem1)
        for j in range(_NSUB)
    ]

    @pl.when(sid == 0)
    def _():
        pltpu.sync_copy(zacc_h, shacc)

    pltpu.sync_copy(dx_h.at[wid], dxv)
    pltpu.sync_copy(dy_h.at[wid], dyv)
    pltpu.sync_copy(zcnt_h, cntv)

    ones = jnp.ones((16,), jnp.float32)

    def row_body(r, carry):
        for q in range(_SUB // 16):
            off = r * _SUB + q * 16
            vdx = dxv[pl.ds(off, 16)]
            vdy = dyv[pl.ds(off, 16)]
            d2 = vdx * vdx + vdy * vdy
            lt1 = (vdx < vdy).astype(jnp.int32)
            le0 = (vdx <= 0.0).astype(jnp.int32)
            lt0 = (vdx < 0.0).astype(jnp.int32)
            le3 = (vdx <= -vdy).astype(jnp.int32)
            lt3 = (vdx < -vdy).astype(jnp.int32)
            sat = ((vdx < 0.0)
                   & (d2 <= vdx * vdx * 1.0000001192092896)).astype(jnp.int32)
            vb = jnp.where(vdy < 0.0, 7 - (lt3 + lt0 + lt1),
                           lt1 + le0 + le3 + sat)
            ub = (d2 >= _R2_EDGES[0]).astype(jnp.int32)
            for t in _R2_EDGES[1:]:
                ub = ub + (d2 >= t).astype(jnp.int32)
            b16 = jnp.where(d2 <= 4.0, ub * 8 + vb, _NB - 1)
            binv[r, pl.ds(q * 16, 16)] = b16
            plsc.addupdate_scatter(cntv, [b16], ones)
        return carry

    lax.fori_loop(0, _NSUB, row_body, None)

    plsc.subcore_barrier()
    scatters = []
    for j in range(_NSUB):
        gathers[j].wait()
        scatters.append(
            pltpu.async_copy(rowsv.at[j], shacc.at[binv.at[j]], sem2,
                             add=True))
    for s in scatters:
        s.wait()
    pltpu.sync_copy(cntv, ocnt_h.at[wid])
    plsc.subcore_barrier()

    @pl.when(sid == 0)
    def _():
        pltpu.sync_copy(shacc, oacc_h.at[cid])


_sc_pool = pl.kernel(
    _sc_pool_body,
    out_type=(
        jax.ShapeDtypeStruct((_NC, _NB, _HID), jnp.float32),
        jax.ShapeDtypeStruct((_NW, _CNTW), jnp.float32),
    ),
    mesh=plsc.VectorSubcoreMesh(core_axis_name="c", subcore_axis_name="s",
                                num_cores=_NC, num_subcores=_NS),
    compiler_params=pltpu.CompilerParams(needs_layout_passes=False,
                                         use_tc_tiling_on_sc=False),
    scratch_types=[
        pltpu.VMEM((_CH,), jnp.float32),
        pltpu.VMEM((_CH,), jnp.float32),
        pltpu.VMEM((_NSUB, _SUB), jnp.int32),
        pltpu.VMEM((_NSUB, _SUB), jnp.int32),
        pltpu.VMEM((_NSUB, _SUB, _HID), jnp.float32),
        pltpu.VMEM((_CNTW,), jnp.float32),
        pltpu.VMEM_SHARED((_NB, _HID), jnp.float32),
        pltpu.SemaphoreType.DMA,
        pltpu.SemaphoreType.DMA,
    ],
)

_REP = np.equal(np.arange(3072)[None, :] // _HID,
                np.arange(64)[:, None]).astype(np.float32)
_EYE = np.eye(_HID, dtype=np.float32)


def _tail_body(p2_ref, pc_ref, rep_ref, wfc_ref, bfc_ref, img_ref, la_ref,
               fv_ref, h_ref, wza_ref, wzv_ref, wzf_ref, whz_ref, zb_ref,
               wna_ref, wnv_ref, wnf_ref, whn_ref, nb_ref, out_ref):
    accf = jnp.sum(p2_ref[...], axis=0, keepdims=True)
    cnt = jnp.sum(pc_ref[...], axis=0, keepdims=True)[:, :64]
    cnt_e = lax.dot_general(cnt, rep_ref[...],
                            (((1,), (0,)), ((), ())),
                            preferred_element_type=jnp.float32)
    scale = jnp.where(cnt_e > 1.0, 1.0 / jnp.maximum(cnt_e, 1.0), 1.0)
    spf = accf * scale
    fsp = lax.dot_general(spf, wfc_ref[...],
                          (((1,), (1,)), ((), ())),
                          preferred_element_type=jnp.float32)
    fsp = jnp.maximum(fsp + bfc_ref[...], 0.0)

    la = la_ref[...]
    v_img = la[0, 0].astype(jnp.int32)
    u_img = 40 - la[0, 1].astype(jnp.int32)
    mu = lax.broadcasted_iota(jnp.int32, (80, 80), 0) == u_img
    mv = lax.broadcasted_iota(jnp.int32, (80, 80), 1) == v_img
    m2 = (mu & mv).astype(jnp.float32)
    t = jnp.sum(img_ref[0] * m2[None, :, :], axis=2)
    fa = jnp.sum(t, axis=1, keepdims=True)

    fv = fv_ref[...]
    h = h_ref[...]
    zx = (jnp.sum(fa * wza_ref[...]) + jnp.sum(fv * wzv_ref[...])
          + jnp.sum(fsp * wzf_ref[...]) + jnp.sum(h * whz_ref[...])
          + zb_ref[0, 0])
    nx = (jnp.sum(fa * wna_ref[...]) + jnp.sum(fv * wnv_ref[...])
          + jnp.sum(fsp * wnf_ref[...]) + jnp.sum(h * whn_ref[...])
          + nb_ref[0, 0])
    zt = 1.0 / (1.0 + jnp.exp(-zx))
    nt = jnp.tanh(nx)
    out_ref[...] = (1.0 - zt) * nt + zt * h


def kernel(loc_agent, loc_others, loc_other_index, feature_img, f_vel,
           hiddens, hidden, W_fc, b_fc, weight_ir, weight_hr, bias_ir,
           bias_hr, weight_iz, weight_hz, bias_iz, bias_hz, weight_in,
           weight_hn, bias_in, bias_hn):
    f32 = jnp.float32
    dx = loc_others[:, 0] - loc_agent[0]
    dy = loc_others[:, 1] - loc_agent[1]
    pad = _NP - _N
    big = jnp.full((pad,), 3.0e9, f32)
    dx2 = jnp.concatenate([dx, big]).reshape(_NW, _CH)
    dy2 = jnp.concatenate([dy, big]).reshape(_NW, _CH)
    idx3 = jnp.concatenate([
        loc_other_index.astype(jnp.int32), jnp.zeros((pad,), jnp.int32)
    ]).reshape(_NW, _NSUB, _SUB)
    zacc = jnp.zeros((_NB, _HID), f32)
    zcnt = jnp.zeros((_CNTW,), f32)

    ht = hiddens.T
    rm = lax.dot_general(ht, jnp.asarray(_EYE), (((0,), (0,)), ((), ())),
                         preferred_element_type=jnp.float32)
    hid1 = lax.optimization_barrier(rm.reshape(_N * _HID))
    hid2 = hid1.reshape(_N, _HID)

    part_acc, part_cnt = _sc_pool(dx2, dy2, idx3, hid2, zacc, zcnt)
    p2 = part_acc.reshape(_NC, _NB * _HID)[:, :64 * _HID]

    img2 = feature_img
    la2 = loc_agent.reshape(1, 2)
    fv2 = f_vel.reshape(1, 16)
    h2 = hidden.reshape(1, _HID)
    wza = weight_iz[:32].reshape(32, 1)
    wzv = weight_iz[32:48].reshape(1, 16)
    wzf = weight_iz[48:].reshape(1, _HID)
    wna = weight_in[:32].reshape(32, 1)
    wnv = weight_in[32:48].reshape(1, 16)
    wnf = weight_in[48:].reshape(1, _HID)
    whz = weight_hz.reshape(1, _HID)
    whn = weight_hn.reshape(1, _HID)
    zb = (bias_iz + bias_hz).reshape(1, 1)
    nb = (bias_in + bias_hn).reshape(1, 1)
    bfc2 = b_fc.reshape(1, _HID)

    ht = pl.pallas_call(
        _tail_body,
        out_shape=jax.ShapeDtypeStruct((1, _HID), jnp.float32),
    )(p2, part_cnt, jnp.asarray(_REP), W_fc, bfc2, img2, la2, fv2, h2,
      wza, wzv, wzf, whz, zb, wna, wnv, wnf, whn, nb)
    return ht.reshape(_HID)


# speedup vs baseline: 1.0656x; 1.0531x over previous
import math

import jax
import jax.numpy as jnp
import numpy as np
from jax import lax
from jax.experimental import pallas as pl
from jax.experimental.pallas import tpu as pltpu
from jax.experimental.pallas import tpu_sc as plsc

_N = 50000
_HID = 48
_NC = 2
_NS = 16
_NW = _NC * _NS
_CH = 1568
_NP = _NW * _CH
_SUB = 112
_NSUB = _CH // _SUB
_NB = 65
_CNTW = 80

_R2_EDGES = tuple((k / 4.0) ** 2 for k in range(1, 8))


def _sc_pool_body(dx_h, dy_h, idx_h, hid_h, zacc_h, zcnt_h, oacc_h, ocnt_h,
                  dxv, dyv, idxv, binv, rowsv, cntv, shacc, sem1, sem2):
    cid = lax.axis_index("c")
    sid = lax.axis_index("s")
    wid = sid * _NC + cid

    pltpu.sync_copy(idx_h.at[wid], idxv)
    gathers = [
        pltpu.async_copy(hid_h.at[idxv.at[j]], rowsv.at[j], sem1)
        for j in range(_NSUB)
    ]

    @pl.when(sid == 0)
    def _():
        pltpu.sync_copy(zacc_h, shacc)

    pltpu.sync_copy(dx_h.at[wid], dxv)
    pltpu.sync_copy(dy_h.at[wid], dyv)
    pltpu.sync_copy(zcnt_h, cntv)

    ones = jnp.ones((16,), jnp.float32)

    def row_body(r, carry):
        for q in range(_SUB // 16):
            off = r * _SUB + q * 16
            vdx = dxv[pl.ds(off, 16)]
            vdy = dyv[pl.ds(off, 16)]
            d2 = vdx * vdx + vdy * vdy
            lt1 = (vdx < vdy).astype(jnp.int32)
            le0 = (vdx <= 0.0).astype(jnp.int32)
            lt0 = (vdx < 0.0).astype(jnp.int32)
            le3 = (vdx <= -vdy).astype(jnp.int32)
            lt3 = (vdx < -vdy).astype(jnp.int32)
            sat = ((vdx < 0.0)
                   & (d2 <= vdx * vdx * 1.0000001192092896)).astype(jnp.int32)
            vb = jnp.where(vdy < 0.0, 7 - (lt3 + lt0 + lt1),
                           lt1 + le0 + le3 + sat)
            ub = (d2 >= _R2_EDGES[0]).astype(jnp.int32)
            for t in _R2_EDGES[1:]:
                ub = ub + (d2 >= t).astype(jnp.int32)
            b16 = jnp.where(d2 <= 4.0, ub * 8 + vb, _NB - 1)
            binv[r, pl.ds(q * 16, 16)] = b16
            plsc.addupdate_scatter(cntv, [b16], ones)
        return carry

    lax.fori_loop(0, _NSUB, row_body, None)

    plsc.subcore_barrier()
    scatters = []
    for j in range(_NSUB):
        gathers[j].wait()
        scatters.append(
            pltpu.async_copy(rowsv.at[j], shacc.at[binv.at[j]], sem2,
                             add=True))
    for s in scatters:
        s.wait()
    pltpu.sync_copy(cntv, ocnt_h.at[wid])
    plsc.subcore_barrier()

    @pl.when(sid == 0)
    def _():
        pltpu.sync_copy(shacc, oacc_h.at[cid])


_sc_pool = pl.kernel(
    _sc_pool_body,
    out_type=(
        jax.ShapeDtypeStruct((_NC, _NB, _HID), jnp.float32),
        jax.ShapeDtypeStruct((_NW, _CNTW), jnp.float32),
    ),
    mesh=plsc.VectorSubcoreMesh(core_axis_name="c", subcore_axis_name="s",
                                num_cores=_NC, num_subcores=_NS),
    compiler_params=pltpu.CompilerParams(needs_layout_passes=False,
                                         use_tc_tiling_on_sc=False),
    scratch_types=[
        pltpu.VMEM((_CH,), jnp.float32),
        pltpu.VMEM((_CH,), jnp.float32),
        pltpu.VMEM((_NSUB, _SUB), jnp.int32),
        pltpu.VMEM((_NSUB, _SUB), jnp.int32),
        pltpu.VMEM((_NSUB, _SUB, _HID), jnp.float32),
        pltpu.VMEM((_CNTW,), jnp.float32),
        pltpu.VMEM_SHARED((_NB, _HID), jnp.float32),
        pltpu.SemaphoreType.DMA,
        pltpu.SemaphoreType.DMA,
    ],
)

_REP = np.equal(np.arange(3072)[None, :] // _HID,
                np.arange(64)[:, None]).astype(np.float32)


def _tail_body(p2_ref, pc_ref, rep_ref, wfc_ref, bfc_ref, img_ref, la_ref,
               fv_ref, h_ref, wza_ref, wzv_ref, wzf_ref, whz_ref, zb_ref,
               wna_ref, wnv_ref, wnf_ref, whn_ref, nb_ref, out_ref):
    accf = jnp.sum(p2_ref[...], axis=0, keepdims=True)
    cnt = jnp.sum(pc_ref[...], axis=0, keepdims=True)[:, :64]
    cnt_e = lax.dot_general(cnt, rep_ref[...],
                            (((1,), (0,)), ((), ())),
                            preferred_element_type=jnp.float32)
    scale = jnp.where(cnt_e > 1.0, 1.0 / jnp.maximum(cnt_e, 1.0), 1.0)
    spf = accf * scale
    fsp = lax.dot_general(spf, wfc_ref[...],
                          (((1,), (1,)), ((), ())),
                          preferred_element_type=jnp.float32)
    fsp = jnp.maximum(fsp + bfc_ref[...], 0.0)

    la = la_ref[...]
    v_img = la[0, 0].astype(jnp.int32)
    u_img = 40 - la[0, 1].astype(jnp.int32)
    mu = lax.broadcasted_iota(jnp.int32, (80, 80), 0) == u_img
    mv = lax.broadcasted_iota(jnp.int32, (80, 80), 1) == v_img
    m2 = (mu & mv).astype(jnp.float32)
    t = jnp.sum(img_ref[0] * m2[None, :, :], axis=2)
    fa = jnp.sum(t, axis=1, keepdims=True)

    fv = fv_ref[...]
    h = h_ref[...]
    zx = (jnp.sum(fa * wza_ref[...]) + jnp.sum(fv * wzv_ref[...])
          + jnp.sum(fsp * wzf_ref[...]) + jnp.sum(h * whz_ref[...])
          + zb_ref[0, 0])
    nx = (jnp.sum(fa * wna_ref[...]) + jnp.sum(fv * wnv_ref[...])
          + jnp.sum(fsp * wnf_ref[...]) + jnp.sum(h * whn_ref[...])
          + nb_ref[0, 0])
    zt = 1.0 / (1.0 + jnp.exp(-zx))
    nt = jnp.tanh(nx)
    out_ref[...] = (1.0 - zt) * nt + zt * h


def kernel(loc_agent, loc_others, loc_other_index, feature_img, f_vel,
           hiddens, hidden, W_fc, b_fc, weight_ir, weight_hr, bias_ir,
           bias_hr, weight_iz, weight_hz, bias_iz, bias_hz, weight_in,
           weight_hn, bias_in, bias_hn):
    f32 = jnp.float32
    dx = loc_others[:, 0] - loc_agent[0]
    dy = loc_others[:, 1] - loc_agent[1]
    pad = _NP - _N
    big = jnp.full((pad,), 3.0e9, f32)
    dx2 = jnp.concatenate([dx, big]).reshape(_NW, _CH)
    dy2 = jnp.concatenate([dy, big]).reshape(_NW, _CH)
    idx3 = jnp.concatenate([
        loc_other_index.astype(jnp.int32), jnp.zeros((pad,), jnp.int32)
    ]).reshape(_NW, _NSUB, _SUB)
    zacc = jnp.zeros((_NB, _HID), f32)
    zcnt = jnp.zeros((_CNTW,), f32)

    hid1 = lax.optimization_barrier(hiddens.reshape(_N * _HID))
    hid2 = hid1.reshape(_N, _HID)

    part_acc, part_cnt = _sc_pool(dx2, dy2, idx3, hid2, zacc, zcnt)
    p2 = part_acc.reshape(_NC, _NB * _HID)[:, :64 * _HID]

    img2 = feature_img
    la2 = loc_agent.reshape(1, 2)
    fv2 = f_vel.reshape(1, 16)
    h2 = hidden.reshape(1, _HID)
    wza = weight_iz[:32].reshape(32, 1)
    wzv = weight_iz[32:48].reshape(1, 16)
    wzf = weight_iz[48:].reshape(1, _HID)
    wna = weight_in[:32].reshape(32, 1)
    wnv = weight_in[32:48].reshape(1, 16)
    wnf = weight_in[48:].reshape(1, _HID)
    whz = weight_hz.reshape(1, _HID)
    whn = weight_hn.reshape(1, _HID)
    zb = (bias_iz + bias_hz).reshape(1, 1)
    nb = (bias_in + bias_hn).reshape(1, 1)
    bfc2 = b_fc.reshape(1, _HID)

    ht = pl.pallas_call(
        _tail_body,
        out_shape=jax.ShapeDtypeStruct((1, _HID), jnp.float32),
    )(p2, part_cnt, jnp.asarray(_REP), W_fc, bfc2, img2, la2, fv2, h2,
      wza, wzv, wzf, whz, zb, wna, wnv, wnf, whn, nb)
    return ht.reshape(_HID)
